# baseline (speedup 1.0000x reference)
import functools

import jax
import jax.numpy as jnp
from jax import lax
from jax.experimental import pallas as pl
from jax.experimental.pallas import tpu as pltpu
from jax.experimental.pallas import tpu_sc as plsc

B = 16384
N_ENT = 1000000
DIM = 64
L = 16
NC, NS = 2, 16
NW = NC * NS
COLS = (N_ENT + 127) // 128
CPW = 256
PAIRS = 4 * B
SELCAP = 8192
BRIDGE_ROWS = PAIRS + NW * L
PW = B // NW

_PARAMS = pltpu.CompilerParams(needs_layout_passes=False)


def _rsqrt(x):
    i = plsc.bitcast(x, jnp.int32)
    i = jnp.full((L,), 0x5F3759DF, jnp.int32) - lax.shift_right_logical(i, 1)
    y = plsc.bitcast(i, jnp.float32)
    for _ in range(3):
        y = y * (1.5 - 0.5 * x * y * y)
    return y


def _iota():
    return lax.iota(jnp.int32, L)


def _scalar(v16):
    return v16[0]


def _lane0():
    return lax.iota(jnp.int32, L) == 0


def _sload(ref, i):
    return plsc.load_gather(ref, [jnp.full((L,), i, jnp.int32)])[0]


def _sstore(ref, i, val, lane0, dtype=jnp.int32):
    plsc.store_scatter(ref, [jnp.full((L,), i, jnp.int32)],
                       jnp.full((L,), val, dtype), mask=lane0)


def _make_k1():
    mesh = plsc.VectorSubcoreMesh(core_axis_name="c", subcore_axis_name="s")

    @functools.partial(
        pl.kernel,
        mesh=mesh,
        compiler_params=_PARAMS,
        out_type=jax.ShapeDtypeStruct((BRIDGE_ROWS, 128), jnp.float32),
        scratch_types=[
            pltpu.VMEM((4096,), jnp.int32),
            pltpu.VMEM((4096,), jnp.int32),
            pltpu.VMEM((4096,), jnp.int32),
            pltpu.VMEM((4096,), jnp.int32),
            pltpu.VMEM((16,), jnp.int32),
            pltpu.VMEM((SELCAP + 16,), jnp.int32),
            pltpu.VMEM((SELCAP + 16,), jnp.int32),
            pltpu.VMEM((SELCAP + 16,), jnp.int32),
            pltpu.VMEM((SELCAP + 16,), jnp.int32),
            pltpu.VMEM((272,), jnp.int32),
            pltpu.VMEM((272,), jnp.int32),
            pltpu.VMEM((64, 128), jnp.float32),
            pltpu.VMEM((64, 128), jnp.float32),
            pltpu.VMEM((144, 128), jnp.float32),
            pltpu.VMEM((128,), jnp.int32),
            pltpu.SemaphoreType.DMA,
            pltpu.SemaphoreType.DMA,
            pltpu.SemaphoreType.DMA,
        ],
    )
    def k1(heads, tails, nheads, ntails, tt, bridge,
           ibuf0, ibuf1, ibuf2, ibuf3, sbuf, sel_e, sel_d, srt_e, srt_d,
           bounds, cur, colbuf0, colbuf1, rowbuf, destbuf, sem0, sem1, sem_s):
        w = lax.axis_index("s") * NC + lax.axis_index("c")
        lo = w * (CPW * 128)
        hi = lo + CPW * 128
        iota = _iota()
        ibufs = (ibuf0, ibuf1, ibuf2, ibuf3)
        SEG = SELCAP // 4

        arrs = (heads, tails, nheads, ntails)
        ns = []
        for r in range(4):
            n_r = jnp.int32(0)
            for half in range(4):
                pltpu.sync_copy(arrs[r].at[pl.ds(half * 4096, 4096)],
                                ibufs[half])

            def scan_body(c, n, r=r):
                half = c >> 8
                cc = c & 255
                v = ibufs[0][pl.ds(cc * L, L)] if False else None
                return n

            def scan_half(half, n, r=r):
                def body(c, n, half=half, r=r):
                    v = ibufs[half][pl.ds(c * L, L)]
                    m = (v >= lo) & (v < hi)
                    plsc.store_compressed(sel_e.at[pl.ds(r * SEG + n, L)],
                                          v, mask=m)
                    d = (r * B + half * 4096) + c * L + iota
                    plsc.store_compressed(sel_d.at[pl.ds(r * SEG + n, L)],
                                          d, mask=m)
                    return n + _scalar(plsc.all_reduce_population_count(m))
                return lax.fori_loop(0, 4096 // L, body, n)

            for half in range(4):
                n_r = scan_half(half, n_r)
            ns.append(n_r)
        n_sel = ns[0] + ns[1] + ns[2] + ns[3]

        for k in range(272 // L):
            cur[pl.ds(k * L, L)] = jnp.zeros((L,), jnp.int32)

        lane0 = _lane0()
        ones = jnp.full((L,), 1, jnp.int32)

        for r in range(4):
            nvec_r = lax.shift_right_logical(ns[r] + (L - 1), 4)

            def hist_body(k, _, r=r, seg_n=ns[r]):
                valid = (k * L + iota) < seg_n
                e16 = sel_e[pl.ds(r * SEG + k * L, L)]
                c16 = jnp.where(valid, ((e16 >> 7) - w * CPW) & 255,
                                256 + iota)
                plsc.addupdate_scatter(cur, [c16], ones)
                return _

            lax.fori_loop(0, nvec_r, hist_body, jnp.int32(0))

        def prefix_body(k, run_v):
            v = cur[pl.ds(k * L, L)]
            s = v
            for step in (1, 2, 4, 8):
                sbuf[pl.ds(0, L)] = s
                sh = plsc.load_gather(sbuf, [jnp.maximum(iota - step, 0)])
                s = s + jnp.where(iota >= step, sh, 0)
            excl = (s - v) + run_v
            bounds[pl.ds(k * L, L)] = excl
            cur[pl.ds(k * L, L)] = excl
            sbuf[pl.ds(0, L)] = s
            tot = plsc.load_gather(sbuf, [jnp.full((L,), 15, jnp.int32)])
            return run_v + tot

        run_v = lax.fori_loop(0, 256 // L, prefix_body,
                              jnp.zeros((L,), jnp.int32))
        plsc.store_scatter(bounds, [jnp.full((L,), 256, jnp.int32)], run_v,
                           mask=lane0)

        for r in range(4):
            nvec_r = lax.shift_right_logical(ns[r] + (L - 1), 4)

            def place_body(k, _, r=r, seg_n=ns[r]):
                i0 = r * SEG + k * L
                valid = (k * L + iota) < seg_n
                e16 = sel_e[pl.ds(i0, L)]
                d16 = sel_d[pl.ds(i0, L)]
                c16 = jnp.where(valid, ((e16 >> 7) - w * CPW) & 255,
                                256 + iota)
                sbuf[pl.ds(0, L)] = c16
                rank = jnp.zeros((L,), jnp.int32)
                for s in range(1, L):
                    sh = plsc.load_gather(sbuf, [jnp.maximum(iota - s, 0)])
                    rank = rank + jnp.where((sh == c16) & (iota >= s), 1, 0)
                p16 = plsc.load_gather(cur, [c16]) + rank
                p16 = jnp.where(valid, jnp.clip(p16, 0, SELCAP - 1),
                                SELCAP + iota)
                plsc.store_scatter(srt_e, [p16], e16)
                plsc.store_scatter(srt_d, [p16], d16)
                plsc.store_scatter(cur, [c16], p16 + 1)
                return _

            lax.fori_loop(0, nvec_r, place_body, jnp.int32(0))

        ncols = jnp.minimum(jnp.int32(CPW), jnp.maximum(jnp.int32(COLS) - w * CPW,
                                                        jnp.int32(0)))
        for k in range(128 // L):
            destbuf[pl.ds(k * L, L)] = PAIRS + w * L + iota

        def issue(j, cb, sem):
            basej = (w * CPW + j) * 128
            pltpu.async_copy(tt.at[:, pl.ds(basej, 128)], cb, sem)

        def drain(j, cb, sem):
            basej = (w * CPW + j) * 128
            pltpu.make_async_copy(tt.at[:, pl.ds(basej, 128)], cb, sem).wait()

        @pl.when(ncols > 0)
        def _():
            issue(jnp.int32(0), colbuf0, sem0)

        @pl.when(ncols > 1)
        def _():
            issue(jnp.int32(1), colbuf1, sem1)

        rows16 = [iota + 16 * k for k in range(4)]

        def serve_col(j, nfill, cb, sem):
            drain(j, cb, sem)
            base = (w * CPW + j) * 128

            lane0s = _lane0()

            def pair_body(p, nf):
                e = _sload(srt_e, p)
                el = (e - base) & 127
                colv = jnp.full((L,), el, jnp.int32)
                nfv = jnp.full((L,), nf, jnp.int32)
                for k in range(4):
                    v = plsc.load_gather(cb, [rows16[k], colv])
                    plsc.store_scatter(rowbuf, [nfv, rows16[k]], v)
                dest = jnp.clip(_sload(srt_d, p), 0, BRIDGE_ROWS - 1)
                _sstore(destbuf, nf, dest, lane0s)
                nf = nf + 1

                @pl.when(nf == 128)
                def _():
                    pltpu.async_copy(rowbuf.at[pl.ds(0, 128)],
                                     bridge.at[destbuf], sem_s).wait()

                return lax.select(nf == 128, jnp.int32(0), nf)

            lo_b = jnp.clip(_sload(bounds, j), 0, n_sel)
            hi_b = jnp.clip(_sload(bounds, j + 1), lo_b, n_sel)
            nfill = lax.fori_loop(lo_b, hi_b, pair_body, nfill)

            @pl.when(j + 2 < ncols)
            def _():
                issue(j + 2, cb, sem)

            return nfill

        def col_body(j, nfill):
            nf0 = lax.cond(j % 2 == 0,
                           lambda nf: serve_col(j, nf, colbuf0, sem0),
                           lambda nf: serve_col(j, nf, colbuf1, sem1),
                           nfill)
            return nf0

        nfill = lax.fori_loop(0, ncols, col_body, jnp.int32(0))

        @pl.when(nfill > 0)
        def _():
            pltpu.async_copy(rowbuf.at[pl.ds(0, 128)], bridge.at[destbuf],
                             sem_s).wait()

    return k1


def _make_k2():
    mesh = plsc.VectorSubcoreMesh(core_axis_name="c", subcore_axis_name="s")

    @functools.partial(
        pl.kernel,
        mesh=mesh,
        compiler_params=_PARAMS,
        out_type=(
            jax.ShapeDtypeStruct((B,), jnp.float32),
            jax.ShapeDtypeStruct((B,), jnp.float32),
        ),
        scratch_types=[
            pltpu.VMEM((500, 128), jnp.float32),
            pltpu.VMEM((PW,), jnp.int32),
            pltpu.VMEM((64, 128), jnp.float32),
            pltpu.VMEM((64, 128), jnp.float32),
            pltpu.VMEM((64, 128), jnp.float32),
            pltpu.VMEM((64, 128), jnp.float32),
            pltpu.VMEM((PW,), jnp.float32),
            pltpu.VMEM((PW,), jnp.float32),
            pltpu.SemaphoreType.DMA,
        ],
    )
    def k2(bridge, relp, relations, out_g, out_n,
           rel_v, ridx, h_v, t_v, nh_v, nt_v, og, on, sem):
        w = lax.axis_index("s") * NC + lax.axis_index("c")
        base = pl.multiple_of(w * PW, PW)
        pltpu.sync_copy(relp, rel_v)
        pltpu.sync_copy(relations.at[pl.ds(base, PW)], ridx)
        iota = _iota()

        def score_pass(a_v, b_v, off, out_ref):
            for g in range(4):
                r16 = iota + g * L
                q = ridx[pl.ds(off + g * L, L)]
                qrow = lax.shift_right_logical(q, 1)
                qcol0 = (q & 1) * 64
                zero = jnp.zeros((L,), jnp.float32)
                hh = zero; tt_ = zero; rr = zero
                hr = zero; ht = zero; rt = zero
                for d in range(DIM):
                    dv = jnp.full((L,), d, jnp.int32)
                    hv = plsc.load_gather(a_v, [r16, dv])
                    tv = plsc.load_gather(b_v, [r16, dv])
                    rv = plsc.load_gather(rel_v, [qrow, qcol0 + dv])
                    hh = hh + hv * hv
                    tt_ = tt_ + tv * tv
                    rr = rr + rv * rv
                    hr = hr + hv * rv
                    ht = ht + hv * tv
                    rt = rt + rv * tv
                ih = _rsqrt(jnp.maximum(hh, 1e-24))
                it = _rsqrt(jnp.maximum(tt_, 1e-24))
                g2 = ((hh * ih) * ih + rr + (tt_ * it) * it
                      + 2.0 * ((hr * ih) - (ht * ih) * it - (rt * it)))
                g2 = jnp.maximum(g2, 0.0)
                res = g2 * _rsqrt(jnp.maximum(g2, 1e-24))
                out_ref[pl.ds(off + g * L, L)] = -res

        def chunk_body(c, carry):
            slot0 = base + c * 64
            a1 = pltpu.async_copy(bridge.at[pl.ds(slot0, 64)], h_v, sem)
            a2 = pltpu.async_copy(bridge.at[pl.ds(B + slot0, 64)], t_v, sem)
            a3 = pltpu.async_copy(bridge.at[pl.ds(2 * B + slot0, 64)], nh_v, sem)
            a4 = pltpu.async_copy(bridge.at[pl.ds(3 * B + slot0, 64)], nt_v, sem)
            a1.wait(); a2.wait(); a3.wait(); a4.wait()
            score_pass(h_v, t_v, c * 64, og)
            score_pass(nh_v, nt_v, c * 64, on)
            return carry

        lax.fori_loop(0, PW // 64, chunk_body, jnp.int32(0))
        pltpu.sync_copy(og, out_g.at[pl.ds(base, PW)])
        pltpu.sync_copy(on, out_n.at[pl.ds(base, PW)])

    return k2


def kernel(heads, tails, negative_heads, negative_tails, relations,
           entity_embeddings, relation_embeddings):
    tt = entity_embeddings.T
    relp = relation_embeddings.reshape(500, 128)
    bridge = _make_k1()(heads, tails, negative_heads, negative_tails, tt)
    return _make_k2()(bridge, relp, relations)


# speedup vs baseline: 1.0306x; 1.0306x over previous
import functools

import jax
import jax.numpy as jnp
from jax import lax
from jax.experimental import pallas as pl
from jax.experimental.pallas import tpu as pltpu
from jax.experimental.pallas import tpu_sc as plsc

B = 16384
N_ENT = 1000000
DIM = 64
L = 16
NC, NS = 2, 16
NW = NC * NS
COLS = (N_ENT + 127) // 128
CPW = 256
PAIRS = 4 * B
SELCAP = 8192
BRIDGE_ROWS = PAIRS + NW * L
PW = B // NW

_PARAMS = pltpu.CompilerParams(needs_layout_passes=False)


def _rsqrt(x):
    i = plsc.bitcast(x, jnp.int32)
    i = jnp.full((L,), 0x5F3759DF, jnp.int32) - lax.shift_right_logical(i, 1)
    y = plsc.bitcast(i, jnp.float32)
    for _ in range(3):
        y = y * (1.5 - 0.5 * x * y * y)
    return y


def _iota():
    return lax.iota(jnp.int32, L)


def _scalar(v16):
    return v16[0]


def _lane0():
    return lax.iota(jnp.int32, L) == 0


def _sload(ref, i):
    return plsc.load_gather(ref, [jnp.full((L,), i, jnp.int32)])[0]


def _sstore(ref, i, val, lane0, dtype=jnp.int32):
    plsc.store_scatter(ref, [jnp.full((L,), i, jnp.int32)],
                       jnp.full((L,), val, dtype), mask=lane0)


def _make_k1():
    mesh = plsc.VectorSubcoreMesh(core_axis_name="c", subcore_axis_name="s")

    @functools.partial(
        pl.kernel,
        mesh=mesh,
        compiler_params=_PARAMS,
        out_type=jax.ShapeDtypeStruct((BRIDGE_ROWS, 128), jnp.float32),
        scratch_types=[
            pltpu.VMEM((4096,), jnp.int32),
            pltpu.VMEM((4096,), jnp.int32),
            pltpu.VMEM((4096,), jnp.int32),
            pltpu.VMEM((4096,), jnp.int32),
            pltpu.VMEM((16,), jnp.int32),
            pltpu.VMEM((SELCAP + 16,), jnp.int32),
            pltpu.VMEM((SELCAP + 16,), jnp.int32),
            pltpu.VMEM((SELCAP + 16,), jnp.int32),
            pltpu.VMEM((SELCAP + 16,), jnp.int32),
            pltpu.VMEM((272,), jnp.int32),
            pltpu.VMEM((272,), jnp.int32),
            pltpu.VMEM((64, 128), jnp.float32),
            pltpu.VMEM((64, 128), jnp.float32),
            pltpu.VMEM((144, 128), jnp.float32),
            pltpu.VMEM((128,), jnp.int32),
            pltpu.SemaphoreType.DMA,
            pltpu.SemaphoreType.DMA,
            pltpu.SemaphoreType.DMA,
        ],
    )
    def k1(heads, tails, nheads, ntails, tt, bridge,
           ibuf0, ibuf1, ibuf2, ibuf3, sbuf, sel_e, sel_d, srt_e, srt_d,
           bounds, cur, colbuf0, colbuf1, rowbuf, destbuf, sem0, sem1, sem_s):
        w = lax.axis_index("s") * NC + lax.axis_index("c")
        lo = w * (CPW * 128)
        hi = lo + CPW * 128
        iota = _iota()
        ibufs = (ibuf0, ibuf1, ibuf2, ibuf3)
        SEG = SELCAP // 4

        arrs = (heads, tails, nheads, ntails)
        ns = []
        for r in range(4):
            n_r = jnp.int32(0)
            for half in range(4):
                pltpu.sync_copy(arrs[r].at[pl.ds(half * 4096, 4096)],
                                ibufs[half])

            def scan_body(c, n, r=r):
                half = c >> 8
                cc = c & 255
                v = ibufs[0][pl.ds(cc * L, L)] if False else None
                return n

            def scan_half(half, n, r=r):
                def body(c, n, half=half, r=r):
                    v = ibufs[half][pl.ds(c * L, L)]
                    m = (v >= lo) & (v < hi)
                    plsc.store_compressed(sel_e.at[pl.ds(r * SEG + n, L)],
                                          v, mask=m)
                    d = (r * B + half * 4096) + c * L + iota
                    plsc.store_compressed(sel_d.at[pl.ds(r * SEG + n, L)],
                                          d, mask=m)
                    return n + _scalar(plsc.all_reduce_population_count(m))
                return lax.fori_loop(0, 4096 // L, body, n)

            for half in range(4):
                n_r = scan_half(half, n_r)
            ns.append(n_r)
        n_sel = ns[0] + ns[1] + ns[2] + ns[3]

        for k in range(272 // L):
            cur[pl.ds(k * L, L)] = jnp.zeros((L,), jnp.int32)

        lane0 = _lane0()
        ones = jnp.full((L,), 1, jnp.int32)

        for r in range(4):
            nvec_r = lax.shift_right_logical(ns[r] + (L - 1), 4)

            def hist_body(k, _, r=r, seg_n=ns[r]):
                valid = (k * L + iota) < seg_n
                e16 = sel_e[pl.ds(r * SEG + k * L, L)]
                c16 = jnp.where(valid, ((e16 >> 7) - w * CPW) & 255,
                                256 + iota)
                plsc.addupdate_scatter(cur, [c16], ones)
                return _

            lax.fori_loop(0, nvec_r, hist_body, jnp.int32(0))

        def prefix_body(k, run_v):
            v = cur[pl.ds(k * L, L)]
            s = v
            for step in (1, 2, 4, 8):
                sbuf[pl.ds(0, L)] = s
                sh = plsc.load_gather(sbuf, [jnp.maximum(iota - step, 0)])
                s = s + jnp.where(iota >= step, sh, 0)
            excl = (s - v) + run_v
            bounds[pl.ds(k * L, L)] = excl
            cur[pl.ds(k * L, L)] = excl
            sbuf[pl.ds(0, L)] = s
            tot = plsc.load_gather(sbuf, [jnp.full((L,), 15, jnp.int32)])
            return run_v + tot

        run_v = lax.fori_loop(0, 256 // L, prefix_body,
                              jnp.zeros((L,), jnp.int32))
        plsc.store_scatter(bounds, [jnp.full((L,), 256, jnp.int32)], run_v,
                           mask=lane0)

        for r in range(4):
            nvec_r = lax.shift_right_logical(ns[r] + (L - 1), 4)

            def place_body(k, _, r=r, seg_n=ns[r]):
                i0 = r * SEG + k * L
                valid = (k * L + iota) < seg_n
                e16 = sel_e[pl.ds(i0, L)]
                d16 = sel_d[pl.ds(i0, L)]
                c16 = jnp.where(valid, ((e16 >> 7) - w * CPW) & 255,
                                256 + iota)
                sbuf[pl.ds(0, L)] = c16
                rank = jnp.zeros((L,), jnp.int32)
                for s in range(1, L):
                    sh = plsc.load_gather(sbuf, [jnp.maximum(iota - s, 0)])
                    rank = rank + jnp.where((sh == c16) & (iota >= s), 1, 0)
                p16 = plsc.load_gather(cur, [c16]) + rank
                p16 = jnp.where(valid, jnp.clip(p16, 0, SELCAP - 1),
                                SELCAP + iota)
                plsc.store_scatter(srt_e, [p16], e16)
                plsc.store_scatter(srt_d, [p16], d16)
                plsc.store_scatter(cur, [c16], p16 + 1)
                return _

            lax.fori_loop(0, nvec_r, place_body, jnp.int32(0))

        ncols = jnp.minimum(jnp.int32(CPW), jnp.maximum(jnp.int32(COLS) - w * CPW,
                                                        jnp.int32(0)))
        for k in range(128 // L):
            destbuf[pl.ds(k * L, L)] = PAIRS + w * L + iota

        def issue(j, cb, sem):
            basej = (w * CPW + j) * 128
            pltpu.async_copy(tt.at[:, pl.ds(basej, 128)], cb, sem)

        def drain(j, cb, sem):
            basej = (w * CPW + j) * 128
            pltpu.make_async_copy(tt.at[:, pl.ds(basej, 128)], cb, sem).wait()

        @pl.when(ncols > 0)
        def _():
            issue(jnp.int32(0), colbuf0, sem0)

        @pl.when(ncols > 1)
        def _():
            issue(jnp.int32(1), colbuf1, sem1)

        rows16 = [iota + 16 * k for k in range(4)]

        def serve_col(j, nfill, cb, sem):
            drain(j, cb, sem)
            base = (w * CPW + j) * 128

            lane0s = _lane0()

            def pair_body(p, nf):
                e = _sload(srt_e, p)
                el = (e - base) & 127
                colv = jnp.full((L,), el, jnp.int32)
                nfv = jnp.full((L,), nf, jnp.int32)
                for k in range(4):
                    v = plsc.load_gather(cb, [rows16[k], colv])
                    plsc.store_scatter(rowbuf, [nfv, rows16[k]], v)
                dest = jnp.clip(_sload(srt_d, p), 0, BRIDGE_ROWS - 1)
                _sstore(destbuf, nf, dest, lane0s)
                nf = nf + 1

                @pl.when(nf == 128)
                def _():
                    pltpu.async_copy(rowbuf.at[pl.ds(0, 128)],
                                     bridge.at[destbuf], sem_s).wait()

                return lax.select(nf == 128, jnp.int32(0), nf)

            lo_b = jnp.clip(_sload(bounds, j), 0, n_sel)
            hi_b = jnp.clip(_sload(bounds, j + 1), lo_b, n_sel)
            nfill = lax.fori_loop(lo_b, hi_b, pair_body, nfill)

            @pl.when(j + 2 < ncols)
            def _():
                issue(j + 2, cb, sem)

            return nfill

        def col_body(j, nfill):
            nf0 = lax.cond(j % 2 == 0,
                           lambda nf: serve_col(j, nf, colbuf0, sem0),
                           lambda nf: serve_col(j, nf, colbuf1, sem1),
                           nfill)
            return nf0

        nfill = lax.fori_loop(0, ncols, col_body, jnp.int32(0))

        @pl.when(nfill > 0)
        def _():
            pltpu.async_copy(rowbuf.at[pl.ds(0, 128)], bridge.at[destbuf],
                             sem_s).wait()

    return k1


def _make_k2():
    mesh = plsc.VectorSubcoreMesh(core_axis_name="c", subcore_axis_name="s")

    @functools.partial(
        pl.kernel,
        mesh=mesh,
        compiler_params=_PARAMS,
        out_type=(
            jax.ShapeDtypeStruct((B,), jnp.float32),
            jax.ShapeDtypeStruct((B,), jnp.float32),
        ),
        scratch_types=[
            pltpu.VMEM((500, 128), jnp.float32),
            pltpu.VMEM((PW,), jnp.int32),
            pltpu.VMEM((32, 128), jnp.float32),
            pltpu.VMEM((32, 128), jnp.float32),
            pltpu.VMEM((32, 128), jnp.float32),
            pltpu.VMEM((32, 128), jnp.float32),
            pltpu.VMEM((32, 128), jnp.float32),
            pltpu.VMEM((32, 128), jnp.float32),
            pltpu.VMEM((32, 128), jnp.float32),
            pltpu.VMEM((32, 128), jnp.float32),
            pltpu.VMEM((PW,), jnp.float32),
            pltpu.VMEM((PW,), jnp.float32),
            pltpu.SemaphoreType.DMA,
            pltpu.SemaphoreType.DMA,
        ],
    )
    def k2(bridge, relp, relations, out_g, out_n,
           rel_v, ridx, h0, t0, nh0, nt0, h1, t1, nh1, nt1,
           og, on, semA, semB):
        w = lax.axis_index("s") * NC + lax.axis_index("c")
        base = pl.multiple_of(w * PW, PW)
        pltpu.sync_copy(relp, rel_v)
        pltpu.sync_copy(relations.at[pl.ds(base, PW)], ridx)
        iota = _iota()

        def score_pass(a_v, b_v, off, out_ref):
            for g in range(2):
                r16 = iota + g * L
                q = ridx[pl.ds(off + g * L, L)]
                qrow = lax.shift_right_logical(q, 1)
                qcol0 = (q & 1) * 64
                zero = jnp.zeros((L,), jnp.float32)
                hh = zero; tt_ = zero; rr = zero
                hr = zero; ht = zero; rt = zero
                for d in range(DIM):
                    dv = jnp.full((L,), d, jnp.int32)
                    hv = plsc.load_gather(a_v, [r16, dv])
                    tv = plsc.load_gather(b_v, [r16, dv])
                    rv = plsc.load_gather(rel_v, [qrow, qcol0 + dv])
                    hh = hh + hv * hv
                    tt_ = tt_ + tv * tv
                    rr = rr + rv * rv
                    hr = hr + hv * rv
                    ht = ht + hv * tv
                    rt = rt + rv * tv
                ih = _rsqrt(jnp.maximum(hh, 1e-24))
                it = _rsqrt(jnp.maximum(tt_, 1e-24))
                g2 = ((hh * ih) * ih + rr + (tt_ * it) * it
                      + 2.0 * ((hr * ih) - (ht * ih) * it - (rt * it)))
                g2 = jnp.maximum(g2, 0.0)
                res = g2 * _rsqrt(jnp.maximum(g2, 1e-24))
                out_ref[pl.ds(off + g * L, L)] = -res

        banks = ((h0, t0, nh0, nt0, semA), (h1, t1, nh1, nt1, semB))

        def issue_chunk(c, bank):
            bh, bt, bnh, bnt, sem = bank
            slot0 = base + c * 32
            pltpu.async_copy(bridge.at[pl.ds(slot0, 32)], bh, sem)
            pltpu.async_copy(bridge.at[pl.ds(B + slot0, 32)], bt, sem)
            pltpu.async_copy(bridge.at[pl.ds(2 * B + slot0, 32)], bnh, sem)
            pltpu.async_copy(bridge.at[pl.ds(3 * B + slot0, 32)], bnt, sem)

        def drain_chunk(c, bank):
            bh, bt, bnh, bnt, sem = bank
            slot0 = base + c * 32
            pltpu.make_async_copy(bridge.at[pl.ds(slot0, 32)], bh, sem).wait()
            pltpu.make_async_copy(bridge.at[pl.ds(B + slot0, 32)], bt,
                                  sem).wait()
            pltpu.make_async_copy(bridge.at[pl.ds(2 * B + slot0, 32)], bnh,
                                  sem).wait()
            pltpu.make_async_copy(bridge.at[pl.ds(3 * B + slot0, 32)], bnt,
                                  sem).wait()

        nchunks = PW // 32
        issue_chunk(jnp.int32(0), banks[0])
        issue_chunk(jnp.int32(1), banks[1])

        def work(c, bank):
            bh, bt, bnh, bnt, _ = bank
            drain_chunk(c, bank)
            score_pass(bh, bt, c * 32, og)
            score_pass(bnh, bnt, c * 32, on)

            @pl.when(c + 2 < nchunks)
            def _():
                issue_chunk(c + 2, bank)

            return jnp.int32(0)

        def chunk_body(c, carry):
            return lax.cond(c % 2 == 0,
                            lambda: work(c, banks[0]),
                            lambda: work(c, banks[1]))

        lax.fori_loop(0, nchunks, chunk_body, jnp.int32(0))
        pltpu.sync_copy(og, out_g.at[pl.ds(base, PW)])
        pltpu.sync_copy(on, out_n.at[pl.ds(base, PW)])

    return k2


def kernel(heads, tails, negative_heads, negative_tails, relations,
           entity_embeddings, relation_embeddings):
    tt = entity_embeddings.T
    relp = relation_embeddings.reshape(500, 128)
    bridge = _make_k1()(heads, tails, negative_heads, negative_tails, tt)
    return _make_k2()(bridge, relp, relations)
